# Initial kernel scaffold; baseline (speedup 1.0000x reference)
import jax
import jax.numpy as jnp
from jax.experimental import pallas as pl


def kernel(x, edge_index, y, W1, b1, W2, b2, W3, b3):
    raise NotImplementedError("write your pallas kernel here")


# speedup vs baseline: 8.4936x; 8.4936x over previous
import functools

import jax
import jax.numpy as jnp
from jax import lax
from jax.experimental import pallas as pl
from jax.experimental.pallas import tpu as pltpu
from jax.experimental.pallas import tpu_sc as plsc

N = 10000
NR = 10240
E = 320000
NSC = 2
NTILE = 16
CHUNK = 128

EPT16 = -(-(E // NTILE) // CHUNK) * CHUNK
NCH16 = EPT16 // CHUNK
EPAD16 = EPT16 * NTILE
EPT32 = -(-(E // (NSC * NTILE)) // CHUNK) * CHUNK
NCH32 = EPT32 // CHUNK
EPAD32 = EPT32 * NSC * NTILE

TROWS = NR // NTILE

BN = 2048
GRID = NR // BN


@functools.lru_cache(maxsize=None)
def _make_agg(hh):
  mesh = plsc.VectorSubcoreMesh(core_axis_name="c", subcore_axis_name="s")

  @functools.partial(
      pl.kernel,
      out_type=jax.ShapeDtypeStruct((NSC, NR, hh), jnp.float32),
      scratch_types=[
          pltpu.VMEM((CHUNK,), jnp.int32),
          pltpu.VMEM((CHUNK,), jnp.int32),
          pltpu.VMEM((CHUNK, hh), jnp.float32),
          pltpu.VMEM_SHARED((NR, hh), jnp.float32),
          pltpu.SemaphoreType.DMA,
      ],
      mesh=mesh,
  )
  def agg(table, srcab, dstp, zrows, out, idxb, dstb, rowb, acc, sem):
    c = lax.axis_index("c")
    s = lax.axis_index("s")
    rbase = pl.multiple_of(s * TROWS, TROWS)
    pltpu.sync_copy(zrows, acc.at[pl.ds(rbase, TROWS)])
    plsc.subcore_barrier()
    ebase = s * EPT16

    def body(k, carry):
      off = pl.multiple_of(ebase + k * CHUNK, CHUNK)
      soff = pl.multiple_of(c * EPAD16 + off, CHUNK)
      pltpu.sync_copy(srcab.at[pl.ds(soff, CHUNK)], idxb)
      pltpu.sync_copy(dstp.at[pl.ds(off, CHUNK)], dstb)
      pltpu.async_copy(table.at[idxb], rowb, sem).wait()
      pltpu.sync_copy(rowb, acc.at[dstb], add=True)
      return carry

    lax.fori_loop(0, NCH16, body, 0)
    plsc.subcore_barrier()
    pltpu.sync_copy(acc.at[pl.ds(rbase, TROWS)], out.at[c, pl.ds(rbase, TROWS)])

  return agg


@functools.lru_cache(maxsize=None)
def _make_agg_part():
  mesh = plsc.VectorSubcoreMesh(core_axis_name="c", subcore_axis_name="s")

  @functools.partial(
      pl.kernel,
      out_type=jax.ShapeDtypeStruct((NSC, NR, 128), jnp.float32),
      scratch_types=[
          pltpu.VMEM((CHUNK,), jnp.int32),
          pltpu.VMEM((CHUNK,), jnp.int32),
          pltpu.VMEM((CHUNK, 128), jnp.float32),
          pltpu.VMEM_SHARED((NR, 128), jnp.float32),
          pltpu.SemaphoreType.DMA,
      ],
      mesh=mesh,
  )
  def agg(table, srcp, dstp, zrows, out, idxb, dstb, rowb, acc, sem):
    c = lax.axis_index("c")
    s = lax.axis_index("s")
    rbase = pl.multiple_of(s * TROWS, TROWS)
    pltpu.sync_copy(zrows, acc.at[pl.ds(rbase, TROWS)])
    plsc.subcore_barrier()
    ebase = (c * NTILE + s) * EPT32

    def body(k, carry):
      off = pl.multiple_of(ebase + k * CHUNK, CHUNK)
      pltpu.sync_copy(srcp.at[pl.ds(off, CHUNK)], idxb)
      pltpu.sync_copy(dstp.at[pl.ds(off, CHUNK)], dstb)
      pltpu.async_copy(table.at[idxb], rowb, sem).wait()
      pltpu.sync_copy(rowb, acc.at[dstb], add=True)
      return carry

    lax.fori_loop(0, NCH32, body, 0)
    plsc.subcore_barrier()
    pltpu.sync_copy(acc.at[pl.ds(rbase, TROWS)], out.at[c, pl.ds(rbase, TROWS)])

  return agg


@functools.lru_cache(maxsize=None)
def _make_deg():
  mesh = plsc.VectorSubcoreMesh(core_axis_name="c", subcore_axis_name="s")

  @functools.partial(
      pl.kernel,
      out_type=jax.ShapeDtypeStruct((NSC, NR, 128), jnp.float32),
      scratch_types=[
          pltpu.VMEM((CHUNK,), jnp.int32),
          pltpu.VMEM((CHUNK, 128), jnp.float32),
          pltpu.VMEM_SHARED((NR, 128), jnp.float32),
      ],
      mesh=mesh,
  )
  def deg(dstp, ones, zrows, out, dstb, onesb, dacc):
    c = lax.axis_index("c")
    s = lax.axis_index("s")
    rbase = pl.multiple_of(s * TROWS, TROWS)
    pltpu.sync_copy(zrows, dacc.at[pl.ds(rbase, TROWS)])
    pltpu.sync_copy(ones, onesb)
    plsc.subcore_barrier()
    ebase = (c * NTILE + s) * EPT32

    def body(k, carry):
      off = pl.multiple_of(ebase + k * CHUNK, CHUNK)
      pltpu.sync_copy(dstp.at[pl.ds(off, CHUNK)], dstb)
      pltpu.sync_copy(onesb, dacc.at[dstb], add=True)
      return carry

    lax.fori_loop(0, NCH32, body, 0)
    plsc.subcore_barrier()
    pltpu.sync_copy(dacc.at[pl.ds(rbase, TROWS)], out.at[c, pl.ds(rbase, TROWS)])

  return deg


def _tc_first(x, w1, degp):
  nprt = degp.shape[1]

  def body(x_ref, w_ref, d_ref, s_ref, dinv_ref):
    d = jnp.sum(d_ref[...], axis=1, keepdims=True) + 1.0
    dinv = lax.rsqrt(d)
    h = jnp.dot(x_ref[...], w_ref[...], preferred_element_type=jnp.float32)
    sv = dinv * h
    s_ref[0] = sv[:, :128]
    s_ref[1] = sv[:, 128:]
    dinv_ref[...] = dinv

  return pl.pallas_call(
      body,
      grid=(GRID,),
      in_specs=[
          pl.BlockSpec((BN, 128), lambda i: (i, 0)),
          pl.BlockSpec((128, 256), lambda i: (0, 0)),
          pl.BlockSpec((BN, nprt), lambda i: (i, 0)),
      ],
      out_specs=[
          pl.BlockSpec((NSC, BN, 128), lambda i: (0, i, 0)),
          pl.BlockSpec((BN, 1), lambda i: (i, 0)),
      ],
      out_shape=[
          jax.ShapeDtypeStruct((NSC, NR, 128), jnp.float32),
          jax.ShapeDtypeStruct((NR, 1), jnp.float32),
      ],
  )(x, w1, degp)


def _tc_mid(agg, sprev, dinv, b, w, hout, split_out):
  hh_in = agg.shape[2]
  hin2 = 2 * hh_in
  hh_out = hout // 2

  def body(a_ref, s_ref, d_ref, b_ref, w_ref, o_ref):
    af = jnp.concatenate([a_ref[0], a_ref[1]], axis=1)
    sf = jnp.concatenate([s_ref[0], s_ref[1]], axis=1)
    dv = d_ref[...]
    h = jnp.maximum(dv * (af + sf) + b_ref[...], 0.0)
    sv = dv * jnp.dot(h, w_ref[...], preferred_element_type=jnp.float32)
    if split_out:
      o_ref[0] = sv[:, :hh_out]
      o_ref[1] = sv[:, hh_out:]
    else:
      o_ref[...] = sv

  if split_out:
    out_spec = pl.BlockSpec((NSC, BN, hh_out), lambda i: (0, i, 0))
    out_shape = jax.ShapeDtypeStruct((NSC, NR, hh_out), jnp.float32)
  else:
    out_spec = pl.BlockSpec((BN, hout), lambda i: (i, 0))
    out_shape = jax.ShapeDtypeStruct((NR, hout), jnp.float32)

  return pl.pallas_call(
      body,
      grid=(GRID,),
      in_specs=[
          pl.BlockSpec((NSC, BN, hh_in), lambda i: (0, i, 0)),
          pl.BlockSpec((NSC, BN, hh_in), lambda i: (0, i, 0)),
          pl.BlockSpec((BN, 1), lambda i: (i, 0)),
          pl.BlockSpec((1, hin2), lambda i: (0, 0)),
          pl.BlockSpec((hin2, hout), lambda i: (0, 0)),
      ],
      out_specs=out_spec,
      out_shape=out_shape,
  )(agg, sprev, dinv, b, w)


def _tc_final(agg, sprev, dinv, b):
  def body(a_ref, s_ref, d_ref, b_ref, o_ref):
    z = d_ref[...] * (a_ref[0] + a_ref[1] + s_ref[...]) + b_ref[...]
    o_ref[...] = jax.nn.sigmoid(jnp.maximum(z, 0.0))

  return pl.pallas_call(
      body,
      grid=(GRID,),
      in_specs=[
          pl.BlockSpec((NSC, BN, 128), lambda i: (0, i, 0)),
          pl.BlockSpec((BN, 128), lambda i: (i, 0)),
          pl.BlockSpec((BN, 1), lambda i: (i, 0)),
          pl.BlockSpec((1, 128), lambda i: (0, 0)),
      ],
      out_specs=pl.BlockSpec((BN, 128), lambda i: (i, 0)),
      out_shape=jax.ShapeDtypeStruct((NR, 128), jnp.float32),
  )(agg, sprev, dinv, b)


@jax.jit
def kernel(x, edge_index, y, W1, b1, W2, b2, W3, b3):
  del y
  src = edge_index[0].astype(jnp.int32)
  dst = edge_index[1].astype(jnp.int32)
  srcp = jnp.zeros((EPAD16,), jnp.int32).at[:E].set(src)
  src_ab = jnp.concatenate([srcp, srcp + NR])
  dstp16 = jnp.full((EPAD16,), N, jnp.int32).at[:E].set(dst)
  srcp32 = jnp.zeros((EPAD32,), jnp.int32).at[:E].set(src)
  dstp32 = jnp.full((EPAD32,), N, jnp.int32).at[:E].set(dst)
  ones128 = jnp.ones((CHUNK, 128), jnp.float32)
  z128 = jnp.zeros((TROWS, 128), jnp.float32)
  xp = jnp.zeros((NR, 128), x.dtype).at[:N].set(x)

  hist = _make_deg()(dstp32, ones128, z128)
  degp = hist[:, :, 0].T

  agg128 = _make_agg(128)
  s1, dinv = _tc_first(xp, W1, degp)
  agg1 = agg128(s1.reshape(NSC * NR, 128), src_ab, dstp16, z128)
  s2 = _tc_mid(agg1, s1, dinv, b1.reshape(1, 256), W2, 256, True)
  agg2 = agg128(s2.reshape(NSC * NR, 128), src_ab, dstp16, z128)
  s3 = _tc_mid(agg2, s2, dinv, b2.reshape(1, 256), W3, 128, False)
  agg3 = _make_agg_part()(s3, srcp32, dstp32, z128)
  out = _tc_final(agg3, s3, dinv, b3.reshape(1, 128))
  return out[:N]
